# baseline (speedup 1.0000x reference)
import functools

import jax
import jax.numpy as jnp
import numpy as np
from jax.experimental import pallas as pl
from jax.experimental.pallas import tpu as pltpu
from jax.sharding import PartitionSpec

B, S, D = 4, 2048, 256
NUM_SLOTS = 32768
ROUTER_DIM = 48
K = 32
QT = 512
CT = 4096
NQ = (B * S) // QT
SCALE = np.float32(1.0 / np.sqrt(ROUTER_DIM))
NEG = np.float32(-np.inf)


def _project_kernel(q_ref, sk_ref, w_ref, rq_ref, rk_ref):
    w = w_ref[...]
    rq_ref[...] = jax.lax.dot_general(
        q_ref[...], w,
        dimension_numbers=(((1,), (1,)), ((), ())),
        preferred_element_type=jnp.float32,
    )
    rk_ref[...] = jax.lax.dot_general(
        sk_ref[...], w,
        dimension_numbers=(((1,), (1,)), ((), ())),
        preferred_element_type=jnp.float32,
    )


def _score_topk_kernel(rq_ref, rk_ref, mask_ref, vals_ref, idx_ref,
                       s_ref, cv_ref, ci_ref, *, nc):
    rq = rq_ref[...]
    NCK = nc * K
    fiota_ct = jax.lax.broadcasted_iota(jnp.int32, (QT, CT), 1).astype(jnp.float32)
    iota_k = jax.lax.broadcasted_iota(jnp.int32, (QT, K), 1)
    fiota_nck = jax.lax.broadcasted_iota(jnp.int32, (QT, NCK), 1).astype(jnp.float32)
    FCT = np.float32(CT)
    FNCK = np.float32(NCK)

    def extract_body(j, ec):
        m, v_, p_ = ec
        sj = s_ref[...]
        fi = jnp.min(jnp.where(sj == m, fiota_ct, FCT), axis=1, keepdims=True)
        sm = jnp.where(fiota_ct == fi, NEG, sj)
        s_ref[...] = sm
        m2 = jnp.max(sm, axis=1, keepdims=True)
        v_ = jnp.where(iota_k == j, m, v_)
        p_ = jnp.where(iota_k == j, fi.astype(jnp.int32), p_)
        return (m2, v_, p_)

    for c in range(nc):
        s = jax.lax.dot_general(
            rq, rk_ref[c * CT:(c + 1) * CT, :],
            dimension_numbers=(((1,), (1,)), ((), ())),
            preferred_element_type=jnp.float32,
        )
        s = s * SCALE + mask_ref[c:c + 1, :]
        s_ref[...] = s
        v0 = jnp.full((QT, K), NEG, jnp.float32)
        p0 = jnp.zeros((QT, K), jnp.int32)
        m0 = jnp.max(s, axis=1, keepdims=True)
        _, v, p = jax.lax.fori_loop(0, K, extract_body, (m0, v0, p0))
        cv_ref[:, c * K:(c + 1) * K] = v
        ci_ref[:, c * K:(c + 1) * K] = p + c * CT

    cidx = ci_ref[...]

    def merge_body(j, mc):
        m, vals_, io_ = mc
        cvj = cv_ref[...]
        ppos = jnp.min(jnp.where(cvj == m, fiota_nck, FNCK), axis=1, keepdims=True)
        hit = fiota_nck == ppos
        slot = jnp.sum(jnp.where(hit, cidx, 0), axis=1, keepdims=True)
        cvm = jnp.where(hit, NEG, cvj)
        cv_ref[...] = cvm
        m2 = jnp.max(cvm, axis=1, keepdims=True)
        vals_ = jnp.where(iota_k == j, m, vals_)
        io_ = jnp.where(iota_k == j, slot, io_)
        return (m2, vals_, io_)

    vals0 = jnp.full((QT, K), NEG, jnp.float32)
    io0 = jnp.zeros((QT, K), jnp.int32)
    mm0 = jnp.max(cv_ref[...], axis=1, keepdims=True)
    _, vals, io = jax.lax.fori_loop(0, K, merge_body, (mm0, vals0, io0))
    vals_ref[...] = vals
    idx_ref[...] = io


def _final_merge_kernel(cv_ref, ci_ref, vals_ref, idx_ref, *, ncand):
    fiota_c = jax.lax.broadcasted_iota(jnp.int32, (QT, ncand), 1).astype(jnp.float32)
    iota_k = jax.lax.broadcasted_iota(jnp.int32, (QT, K), 1)
    fncand = np.float32(ncand)
    cv = cv_ref[...]
    cidx = ci_ref[...]

    def body(j, mc):
        cv_, m, vals_, io_ = mc
        ppos = jnp.min(jnp.where(cv_ == m, fiota_c, fncand), axis=1, keepdims=True)
        hit = fiota_c == ppos
        slot = jnp.sum(jnp.where(hit, cidx, 0), axis=1, keepdims=True)
        cv_ = jnp.where(hit, NEG, cv_)
        m2 = jnp.max(cv_, axis=1, keepdims=True)
        vals_ = jnp.where(iota_k == j, m, vals_)
        io_ = jnp.where(iota_k == j, slot, io_)
        return (cv_, m2, vals_, io_)

    vals0 = jnp.full((QT, K), NEG, jnp.float32)
    io0 = jnp.zeros((QT, K), jnp.int32)
    m0 = jnp.max(cv, axis=1, keepdims=True)
    _, _, vals, io = jax.lax.fori_loop(0, K, body, (cv, m0, vals0, io0))
    vals_ref[...] = vals
    idx_ref[...] = io


def _shard_topk(q_flat, sk, mask, w, *, nc):
    loc = nc * CT
    rq, rk = pl.pallas_call(
        _project_kernel,
        grid=(NQ,),
        in_specs=[
            pl.BlockSpec((QT, D), lambda i: (i, 0)),
            pl.BlockSpec((loc // NQ, D), lambda i: (i, 0)),
            pl.BlockSpec((ROUTER_DIM, D), lambda i: (0, 0)),
        ],
        out_specs=[
            pl.BlockSpec((QT, ROUTER_DIM), lambda i: (i, 0)),
            pl.BlockSpec((loc // NQ, ROUTER_DIM), lambda i: (i, 0)),
        ],
        out_shape=[
            jax.ShapeDtypeStruct((B * S, ROUTER_DIM), jnp.float32),
            jax.ShapeDtypeStruct((loc, ROUTER_DIM), jnp.float32),
        ],
    )(q_flat, sk, w)

    vals, idx = pl.pallas_call(
        functools.partial(_score_topk_kernel, nc=nc),
        grid=(NQ,),
        in_specs=[
            pl.BlockSpec((QT, ROUTER_DIM), lambda qi: (qi, 0)),
            pl.BlockSpec((loc, ROUTER_DIM), lambda qi: (0, 0)),
            pl.BlockSpec((nc, CT), lambda qi: (0, 0)),
        ],
        out_specs=[
            pl.BlockSpec((QT, K), lambda qi: (qi, 0)),
            pl.BlockSpec((QT, K), lambda qi: (qi, 0)),
        ],
        out_shape=[
            jax.ShapeDtypeStruct((B * S, K), jnp.float32),
            jax.ShapeDtypeStruct((B * S, K), jnp.int32),
        ],
        scratch_shapes=[
            pltpu.VMEM((QT, CT), jnp.float32),
            pltpu.VMEM((QT, nc * K), jnp.float32),
            pltpu.VMEM((QT, nc * K), jnp.int32),
        ],
    )(rq, rk, mask.reshape(nc, CT))

    shard = jax.lax.axis_index("x")
    idx = idx + (shard * loc).astype(jnp.int32)
    return vals, idx


def _make_kernel():
    devs = jax.devices()
    ndev = len(devs)
    while ndev > 1 and (NUM_SLOTS % (ndev * CT) != 0 or ndev & (ndev - 1)):
        ndev -= 1
    nc = NUM_SLOTS // (ndev * CT)
    mesh = jax.make_mesh((ndev,), ("x",), devices=devs[:ndev])
    P = PartitionSpec

    sharded = jax.shard_map(
        functools.partial(_shard_topk, nc=nc),
        mesh=mesh,
        in_specs=(P(), P("x"), P("x"), P()),
        out_specs=(P(None, "x"), P(None, "x")),
        check_vma=False,
    )

    def _sh(spec):
        return jax.sharding.NamedSharding(mesh, spec)

    ncand = ndev * K
    nrows_loc = (B * S) // ndev

    def _merge_local(cv, ci):
        return pl.pallas_call(
            functools.partial(_final_merge_kernel, ncand=ncand),
            grid=(nrows_loc // QT,),
            in_specs=[
                pl.BlockSpec((QT, ncand), lambda qi: (qi, 0)),
                pl.BlockSpec((QT, ncand), lambda qi: (qi, 0)),
            ],
            out_specs=[
                pl.BlockSpec((QT, K), lambda qi: (qi, 0)),
                pl.BlockSpec((QT, K), lambda qi: (qi, 0)),
            ],
            out_shape=[
                jax.ShapeDtypeStruct((nrows_loc, K), jnp.float32),
                jax.ShapeDtypeStruct((nrows_loc, K), jnp.int32),
            ],
        )(cv, ci)

    merge_sharded = jax.shard_map(
        _merge_local,
        mesh=mesh,
        in_specs=(P("x"), P("x")),
        out_specs=(P("x"), P("x")),
        check_vma=False,
    )

    @jax.jit
    def kernel_fn(query, slot_keys, reliability_mask, W_router):
        q_flat = jax.reshard(query.reshape(B * S, D), _sh(P()))
        sk = jax.reshard(slot_keys, _sh(P("x")))
        rmask = jax.reshard(reliability_mask, _sh(P("x")))
        w = jax.reshard(W_router, _sh(P()))
        cand_vals, cand_idx = sharded(q_flat, sk, rmask, w)
        cand_vals = jax.reshard(cand_vals, _sh(P("x")))
        cand_idx = jax.reshard(cand_idx, _sh(P("x")))
        top_vals, top_idx = merge_sharded(cand_vals, cand_idx)
        return (top_idx.reshape(B, S, K), top_vals.reshape(B, S, K))

    return kernel_fn


_kernel_impl = None


def kernel(query, slot_keys, reliability_mask, W_router):
    global _kernel_impl
    if _kernel_impl is None:
        _kernel_impl = _make_kernel()
    return _kernel_impl(query, slot_keys, reliability_mask, W_router)


# speedup vs baseline: 1.0241x; 1.0241x over previous
import functools

import jax
import jax.numpy as jnp
import numpy as np
from jax.experimental import pallas as pl
from jax.experimental.pallas import tpu as pltpu
from jax.sharding import PartitionSpec

B, S, D = 4, 2048, 256
NUM_SLOTS = 32768
ROUTER_DIM = 48
K = 32
QT = 512
CT = 2048
NQ = (B * S) // QT
SCALE = np.float32(1.0 / np.sqrt(ROUTER_DIM))
NEG = np.float32(-np.inf)


def _project_kernel(q_ref, sk_ref, w_ref, rq_ref, rk_ref):
    w = w_ref[...]
    rq_ref[...] = jax.lax.dot_general(
        q_ref[...], w,
        dimension_numbers=(((1,), (1,)), ((), ())),
        preferred_element_type=jnp.float32,
    )
    rk_ref[...] = jax.lax.dot_general(
        sk_ref[...], w,
        dimension_numbers=(((1,), (1,)), ((), ())),
        preferred_element_type=jnp.float32,
    )


def _score_topk_kernel(rq_ref, rk_ref, mask_ref, vals_ref, idx_ref,
                       s_ref, cv_ref, ci_ref, *, nc):
    rq = rq_ref[...]
    NCK = nc * K
    fiota_ct = jax.lax.broadcasted_iota(jnp.int32, (QT, CT), 1).astype(jnp.float32)
    iota_k = jax.lax.broadcasted_iota(jnp.int32, (QT, K), 1)
    fiota_nck = jax.lax.broadcasted_iota(jnp.int32, (QT, NCK), 1).astype(jnp.float32)
    FCT = np.float32(CT)
    FNCK = np.float32(NCK)

    def extract_body(j, ec):
        m, v_, p_ = ec
        sj = s_ref[...]
        fi = jnp.min(jnp.where(sj == m, fiota_ct, FCT), axis=1, keepdims=True)
        sm = jnp.where(fiota_ct == fi, NEG, sj)
        s_ref[...] = sm
        m2 = jnp.max(sm, axis=1, keepdims=True)
        v_ = jnp.where(iota_k == j, m, v_)
        p_ = jnp.where(iota_k == j, fi.astype(jnp.int32), p_)
        return (m2, v_, p_)

    for c in range(nc):
        s = jax.lax.dot_general(
            rq, rk_ref[c * CT:(c + 1) * CT, :],
            dimension_numbers=(((1,), (1,)), ((), ())),
            preferred_element_type=jnp.float32,
        )
        s = s * SCALE + mask_ref[c:c + 1, :]
        s_ref[...] = s
        v0 = jnp.full((QT, K), NEG, jnp.float32)
        p0 = jnp.zeros((QT, K), jnp.int32)
        m0 = jnp.max(s, axis=1, keepdims=True)
        _, v, p = jax.lax.fori_loop(0, K, extract_body, (m0, v0, p0))
        cv_ref[:, c * K:(c + 1) * K] = v
        ci_ref[:, c * K:(c + 1) * K] = p + c * CT

    cidx = ci_ref[...]

    def merge_body(j, mc):
        m, vals_, io_ = mc
        cvj = cv_ref[...]
        ppos = jnp.min(jnp.where(cvj == m, fiota_nck, FNCK), axis=1, keepdims=True)
        hit = fiota_nck == ppos
        slot = jnp.sum(jnp.where(hit, cidx, 0), axis=1, keepdims=True)
        cvm = jnp.where(hit, NEG, cvj)
        cv_ref[...] = cvm
        m2 = jnp.max(cvm, axis=1, keepdims=True)
        vals_ = jnp.where(iota_k == j, m, vals_)
        io_ = jnp.where(iota_k == j, slot, io_)
        return (m2, vals_, io_)

    vals0 = jnp.full((QT, K), NEG, jnp.float32)
    io0 = jnp.zeros((QT, K), jnp.int32)
    mm0 = jnp.max(cv_ref[...], axis=1, keepdims=True)
    _, vals, io = jax.lax.fori_loop(0, K, merge_body, (mm0, vals0, io0))
    vals_ref[...] = vals
    idx_ref[...] = io


def _final_merge_kernel(cv_ref, ci_ref, vals_ref, idx_ref, *, ncand):
    fiota_c = jax.lax.broadcasted_iota(jnp.int32, (QT, ncand), 1).astype(jnp.float32)
    iota_k = jax.lax.broadcasted_iota(jnp.int32, (QT, K), 1)
    fncand = np.float32(ncand)
    cv = cv_ref[...]
    cidx = ci_ref[...]

    def body(j, mc):
        cv_, m, vals_, io_ = mc
        ppos = jnp.min(jnp.where(cv_ == m, fiota_c, fncand), axis=1, keepdims=True)
        hit = fiota_c == ppos
        slot = jnp.sum(jnp.where(hit, cidx, 0), axis=1, keepdims=True)
        cv_ = jnp.where(hit, NEG, cv_)
        m2 = jnp.max(cv_, axis=1, keepdims=True)
        vals_ = jnp.where(iota_k == j, m, vals_)
        io_ = jnp.where(iota_k == j, slot, io_)
        return (cv_, m2, vals_, io_)

    vals0 = jnp.full((QT, K), NEG, jnp.float32)
    io0 = jnp.zeros((QT, K), jnp.int32)
    m0 = jnp.max(cv, axis=1, keepdims=True)
    _, _, vals, io = jax.lax.fori_loop(0, K, body, (cv, m0, vals0, io0))
    vals_ref[...] = vals
    idx_ref[...] = io


def _shard_topk(q_flat, sk, mask, w, *, nc):
    loc = nc * CT
    rq, rk = pl.pallas_call(
        _project_kernel,
        grid=(NQ,),
        in_specs=[
            pl.BlockSpec((QT, D), lambda i: (i, 0)),
            pl.BlockSpec((loc // NQ, D), lambda i: (i, 0)),
            pl.BlockSpec((ROUTER_DIM, D), lambda i: (0, 0)),
        ],
        out_specs=[
            pl.BlockSpec((QT, ROUTER_DIM), lambda i: (i, 0)),
            pl.BlockSpec((loc // NQ, ROUTER_DIM), lambda i: (i, 0)),
        ],
        out_shape=[
            jax.ShapeDtypeStruct((B * S, ROUTER_DIM), jnp.float32),
            jax.ShapeDtypeStruct((loc, ROUTER_DIM), jnp.float32),
        ],
    )(q_flat, sk, w)

    vals, idx = pl.pallas_call(
        functools.partial(_score_topk_kernel, nc=nc),
        grid=(NQ,),
        in_specs=[
            pl.BlockSpec((QT, ROUTER_DIM), lambda qi: (qi, 0)),
            pl.BlockSpec((loc, ROUTER_DIM), lambda qi: (0, 0)),
            pl.BlockSpec((nc, CT), lambda qi: (0, 0)),
        ],
        out_specs=[
            pl.BlockSpec((QT, K), lambda qi: (qi, 0)),
            pl.BlockSpec((QT, K), lambda qi: (qi, 0)),
        ],
        out_shape=[
            jax.ShapeDtypeStruct((B * S, K), jnp.float32),
            jax.ShapeDtypeStruct((B * S, K), jnp.int32),
        ],
        scratch_shapes=[
            pltpu.VMEM((QT, CT), jnp.float32),
            pltpu.VMEM((QT, nc * K), jnp.float32),
            pltpu.VMEM((QT, nc * K), jnp.int32),
        ],
    )(rq, rk, mask.reshape(nc, CT))

    shard = jax.lax.axis_index("x")
    idx = idx + (shard * loc).astype(jnp.int32)
    return vals, idx


def _make_kernel():
    devs = jax.devices()
    ndev = len(devs)
    while ndev > 1 and (NUM_SLOTS % (ndev * CT) != 0 or ndev & (ndev - 1)):
        ndev -= 1
    nc = NUM_SLOTS // (ndev * CT)
    mesh = jax.make_mesh((ndev,), ("x",), devices=devs[:ndev])
    P = PartitionSpec

    sharded = jax.shard_map(
        functools.partial(_shard_topk, nc=nc),
        mesh=mesh,
        in_specs=(P(), P("x"), P("x"), P()),
        out_specs=(P(None, "x"), P(None, "x")),
        check_vma=False,
    )

    def _sh(spec):
        return jax.sharding.NamedSharding(mesh, spec)

    ncand = ndev * K
    nrows_loc = (B * S) // ndev

    def _merge_local(cv, ci):
        return pl.pallas_call(
            functools.partial(_final_merge_kernel, ncand=ncand),
            grid=(nrows_loc // QT,),
            in_specs=[
                pl.BlockSpec((QT, ncand), lambda qi: (qi, 0)),
                pl.BlockSpec((QT, ncand), lambda qi: (qi, 0)),
            ],
            out_specs=[
                pl.BlockSpec((QT, K), lambda qi: (qi, 0)),
                pl.BlockSpec((QT, K), lambda qi: (qi, 0)),
            ],
            out_shape=[
                jax.ShapeDtypeStruct((nrows_loc, K), jnp.float32),
                jax.ShapeDtypeStruct((nrows_loc, K), jnp.int32),
            ],
        )(cv, ci)

    merge_sharded = jax.shard_map(
        _merge_local,
        mesh=mesh,
        in_specs=(P("x"), P("x")),
        out_specs=(P("x"), P("x")),
        check_vma=False,
    )

    @jax.jit
    def kernel_fn(query, slot_keys, reliability_mask, W_router):
        q_flat = jax.reshard(query.reshape(B * S, D), _sh(P()))
        sk = jax.reshard(slot_keys, _sh(P("x")))
        rmask = jax.reshard(reliability_mask, _sh(P("x")))
        w = jax.reshard(W_router, _sh(P()))
        cand_vals, cand_idx = sharded(q_flat, sk, rmask, w)
        cand_vals = jax.reshard(cand_vals, _sh(P("x")))
        cand_idx = jax.reshard(cand_idx, _sh(P("x")))
        top_vals, top_idx = merge_sharded(cand_vals, cand_idx)
        return (top_idx.reshape(B, S, K), top_vals.reshape(B, S, K))

    return kernel_fn


_kernel_impl = None


def kernel(query, slot_keys, reliability_mask, W_router):
    global _kernel_impl
    if _kernel_impl is None:
        _kernel_impl = _make_kernel()
    return _kernel_impl(query, slot_keys, reliability_mask, W_router)
